# baseline (speedup 1.0000x reference)
import functools
import math

import jax
import jax.numpy as jnp
from jax import lax
from jax.experimental import pallas as pl
from jax.experimental.pallas import tpu as pltpu
from jax.experimental.pallas import tpu_sc as plsc

VOCAB = 1000000
D_MODEL = 64
SCALE = math.sqrt(D_MODEL)

B = 4096
L = 200
B_TOTAL = B * L
NUM_WORKERS = 32
G = 128
GRPS_PER_W = B_TOTAL // (NUM_WORKERS * G)
LANES = 16
R_TILES = D_MODEL // 8

NBUF_I = 4
NBUF_O = 2
T_OUTER = GRPS_PER_W // NBUF_I


def _body(x_hbm, w_hbm, out_hbm, idx_v, in_rows, tbuf, gsem, wsem):
    nc = 2
    wid = lax.axis_index("s") * nc + lax.axis_index("c")
    gid0 = wid * GRPS_PER_W

    pltpu.sync_copy(x_hbm.at[pl.ds(gid0 * G, GRPS_PER_W * G)], idx_v)

    iota16 = lax.iota(jnp.int32, LANES)
    zeros16 = iota16 * 0
    perms = [(iota16 + d) & 15 for d in range(LANES)]
    pre_src = [p * D_MODEL + iota16 for p in perms]
    brv0 = (((iota16 >> 3) * 16) + (iota16 & 7)) * G
    pre_dst = [brv0 + p for p in perms]

    def gather_start(t, bi):
        pltpu.async_copy(
            w_hbm.at[idx_v.at[pl.ds(t * G, G)]], in_rows.at[bi], gsem.at[bi])

    def gather_wait(bi):
        pltpu.make_async_copy(
            w_hbm.at[idx_v.at[pl.ds(0, G)]], in_rows.at[bi], gsem.at[bi]).wait()

    def write_start(t, bo):
        gidm = gid0 + t - 1
        l = gidm >> 5
        c0 = gidm & 31
        for r in range(R_TILES):
            pltpu.async_copy(
                tbuf.at[bo, pl.ds(r * 16, 16)],
                out_hbm.at[l, r, pl.ds(c0 * 8, 16)], wsem.at[bo])

    def write_wait(bo):
        for r in range(R_TILES):
            pltpu.make_async_copy(
                tbuf.at[bo, pl.ds(r * 16, 16)],
                out_hbm.at[0, r, pl.ds(0, 16)], wsem.at[bo]).wait()

    def transpose_scale(bi, bo, csub):
        src = in_rows.at[bi]
        dst = tbuf.at[bo]

        def block(k, _):
            sbase = k * (LANES * D_MODEL)
            dbase = k * LANES + csub * 1024
            for j in range(D_MODEL // LANES):
                s_off = sbase + LANES * j
                d_off = dbase + 4096 * j
                for d in range(LANES):
                    vals = plsc.load_gather(src, [zeros16, pre_src[d] + s_off])
                    plsc.store_scatter(
                        dst, [zeros16, pre_dst[d] + d_off], vals)
            return 0

        lax.fori_loop(0, G // LANES, block, 0)

    for b in range(NBUF_I):
        gather_start(b, b)

    def step(tt, _):
        for q in range(NBUF_I):
            t = tt * NBUF_I + q
            bo = q >> 1
            csub = q & 1
            gather_wait(q)
            if csub == 0:
                @pl.when(tt > 0)
                def _():
                    write_wait(bo)
            transpose_scale(q, bo, csub)

            @pl.when(tt < T_OUTER - 1)
            def _():
                gather_start(t + NBUF_I, q)
            if csub == 1:
                write_start(t, bo)
        return 0

    lax.fori_loop(0, T_OUTER, step, 0)

    for bo in range(NBUF_O):
        write_wait(bo)


@jax.jit
def _embed(x_lmajor, weight):
    mesh = plsc.VectorSubcoreMesh(core_axis_name="c", subcore_axis_name="s")
    kfn = pl.kernel(
        _body,
        mesh=mesh,
        out_type=jax.ShapeDtypeStruct((L, R_TILES, 256, G), jnp.float32),
        scratch_types=[
            pltpu.VMEM((GRPS_PER_W * G,), jnp.int32),
            pltpu.VMEM((NBUF_I, G, D_MODEL), jnp.float32),
            pltpu.VMEM((NBUF_O, G, G), jnp.float32),
            pltpu.SemaphoreType.DMA((NBUF_I,)),
            pltpu.SemaphoreType.DMA((NBUF_O,)),
        ],
        compiler_params=pltpu.CompilerParams(
            use_tc_tiling_on_sc=False, needs_layout_passes=False),
    )
    return kfn(x_lmajor, weight)


def kernel(x, weight):
    xin = x.T.reshape(B_TOTAL)
    out3 = _embed(xin, weight * SCALE)
    out5 = out3.reshape(L, R_TILES, 32, 8, G)
    return out5.transpose(2, 4, 0, 1, 3).reshape(B, L, D_MODEL)


# speedup vs baseline: 1.2516x; 1.2516x over previous
import functools
import math

import jax
import jax.numpy as jnp
from jax import lax
from jax.experimental import pallas as pl
from jax.experimental.pallas import tpu as pltpu
from jax.experimental.pallas import tpu_sc as plsc

VOCAB = 1000000
D_MODEL = 64
SCALE = math.sqrt(D_MODEL)

B = 4096
L = 200
B_TOTAL = B * L
NUM_WORKERS = 32
G = 128
GRPS_PER_W = B_TOTAL // (NUM_WORKERS * G)
LANES = 16
R_TILES = D_MODEL // 8

NBUF_I = 4
NBUF_O = 2
T_OUTER = GRPS_PER_W // NBUF_I


def _body(x_hbm, w_hbm, out_hbm, idx_v, in_rows, tbuf, gsem, wsem):
    nc = 2
    wid = lax.axis_index("s") * nc + lax.axis_index("c")
    gid0 = wid * GRPS_PER_W

    pltpu.sync_copy(x_hbm.at[pl.ds(gid0 * G, GRPS_PER_W * G)], idx_v)

    iota16 = lax.iota(jnp.int32, LANES)
    zeros16 = iota16 * 0
    perms = [(iota16 + d) & 15 for d in range(LANES)]
    pre_src = [p * D_MODEL + iota16 for p in perms]
    brv0 = (((iota16 >> 3) * 16) + (iota16 & 7)) * G
    pre_dst = [brv0 + p for p in perms]

    def gather_start(t, bi):
        pltpu.async_copy(
            w_hbm.at[idx_v.at[pl.ds(t * G, G)]], in_rows.at[bi], gsem.at[bi])

    def gather_wait(bi):
        pltpu.make_async_copy(
            w_hbm.at[idx_v.at[pl.ds(0, G)]], in_rows.at[bi], gsem.at[bi]).wait()

    def write_start(t, bo):
        gidm = gid0 + t - 1
        l = gidm >> 5
        c0 = gidm & 31
        for r in range(R_TILES):
            pltpu.async_copy(
                tbuf.at[bo, pl.ds(r * 16, 16)],
                out_hbm.at[l, r, pl.ds(c0 * 8, 16)], wsem.at[bo])

    def write_wait(bo):
        for r in range(R_TILES):
            pltpu.make_async_copy(
                tbuf.at[bo, pl.ds(r * 16, 16)],
                out_hbm.at[0, r, pl.ds(0, 16)], wsem.at[bo]).wait()

    def transpose_scale(bi, bo, csub):
        src = in_rows.at[bi]
        dst = tbuf.at[bo]

        def block(k, _):
            sbase = k * (LANES * D_MODEL)
            dbase = k * LANES + csub * 1024
            for j in range(D_MODEL // LANES):
                s_off = sbase + LANES * j
                d_off = dbase + 4096 * j
                for d in range(LANES):
                    vals = plsc.load_gather(src, [zeros16, pre_src[d] + s_off])
                    plsc.store_scatter(
                        dst, [zeros16, pre_dst[d] + d_off], vals * SCALE)
            return 0

        lax.fori_loop(0, G // LANES, block, 0)

    for b in range(NBUF_I):
        gather_start(b, b)

    def step(tt, _):
        for q in range(NBUF_I):
            t = tt * NBUF_I + q
            bo = q >> 1
            csub = q & 1
            gather_wait(q)
            if csub == 0:
                @pl.when(tt > 0)
                def _():
                    write_wait(bo)
            transpose_scale(q, bo, csub)

            @pl.when(tt < T_OUTER - 1)
            def _():
                gather_start(t + NBUF_I, q)
            if csub == 1:
                write_start(t, bo)
        return 0

    lax.fori_loop(0, T_OUTER, step, 0)

    for bo in range(NBUF_O):
        write_wait(bo)


@jax.jit
def _embed(x_lmajor, weight):
    mesh = plsc.VectorSubcoreMesh(core_axis_name="c", subcore_axis_name="s")
    kfn = pl.kernel(
        _body,
        mesh=mesh,
        out_type=jax.ShapeDtypeStruct((L, R_TILES, 256, G), jnp.float32),
        scratch_types=[
            pltpu.VMEM((GRPS_PER_W * G,), jnp.int32),
            pltpu.VMEM((NBUF_I, G, D_MODEL), jnp.float32),
            pltpu.VMEM((NBUF_O, G, G), jnp.float32),
            pltpu.SemaphoreType.DMA((NBUF_I,)),
            pltpu.SemaphoreType.DMA((NBUF_O,)),
        ],
        compiler_params=pltpu.CompilerParams(
            use_tc_tiling_on_sc=False, needs_layout_passes=False),
    )
    return kfn(x_lmajor, weight)


def kernel(x, weight):
    xin = x.T.reshape(B_TOTAL)
    out3 = _embed(xin, weight)
    out5 = out3.reshape(L, R_TILES, 32, 8, G)
    return out5.transpose(2, 4, 0, 1, 3).reshape(B, L, D_MODEL)
